# baseline (speedup 1.0000x reference)
import functools

import jax
import jax.numpy as jnp
from jax import lax
from jax.experimental import pallas as pl
from jax.experimental.pallas import tpu as pltpu
from jax.experimental.pallas import tpu_sc as plsc

N = 10000
E = 320000
H = 4
C = 128
DE = H * C
D_IN = 128

NSUB = 16
BLK = 96
NBLK = 208
EPS_MAIN = NBLK * BLK
TAIL = 32
E_MAIN = NSUB * EPS_MAIN
NPAIR = NBLK // 2
NPAD = 10240
ROWS_PER_SUB = NPAD // NSUB


def _stage1_body(x_ref, w_ref, a_ref, h_ref, alphas_ref):
    h = jnp.dot(x_ref[...], w_ref[...], preferred_element_type=jnp.float32)
    h_ref[...] = h
    alphas_ref[...] = jnp.dot(h, a_ref[...],
                              preferred_element_type=jnp.float32)


def _stage1(x, W, A):
    blk = 1000
    grid = N // blk
    return pl.pallas_call(
        _stage1_body,
        grid=(grid,),
        in_specs=[
            pl.BlockSpec((blk, D_IN), lambda i: (i, 0)),
            pl.BlockSpec((D_IN, DE), lambda i: (0, 0)),
            pl.BlockSpec((DE, 2 * H), lambda i: (0, 0)),
        ],
        out_specs=[
            pl.BlockSpec((blk, DE), lambda i: (i, 0)),
            pl.BlockSpec((blk, 2 * H), lambda i: (i, 0)),
        ],
        out_shape=[
            jax.ShapeDtypeStruct((N, DE), jnp.float32),
            jax.ShapeDtypeStruct((N, 2 * H), jnp.float32),
        ],
    )(x, W, A)


def _sc_body(htable, e2, alphasP,
             msg_out, denomp_out,
             acc, apk_v, denom_v,
             eb0, eb1, ridx0, ridx1, w0, w1, dc0, dc1,
             rows0, rows1, teb, tridx, tw, tdc,
             semi0, semi1, semg0, semg1, sems0, sems1):
    c = lax.axis_index("c")
    s = lax.axis_index("s")
    EB = (eb0, eb1)
    RIDX = (ridx0, ridx1)
    WV = (w0, w1)
    DC = (dc0, dc1)
    ROWS = (rows0, rows1)
    SEMI = (semi0, semi1)
    SEMG = (semg0, semg1)
    SEMS = (sems0, sems1)
    row0 = pl.multiple_of(s * ROWS_PER_SUB, 8)
    bbase = s * NBLK
    zero16 = jnp.zeros((16,), jnp.float32)
    himask = jnp.full((16,), -65536, jnp.int32)

    def issue_idx(block, par):
        off = pl.multiple_of((bbase + block) * (2 * BLK), 8)
        pltpu.async_copy(e2.at[pl.ds(off, 2 * BLK)], EB[par], SEMI[par])

    def wait_idx(par):
        pltpu.make_async_copy(e2.at[pl.ds(0, 2 * BLK)], EB[par],
                              SEMI[par]).wait()

    def weights(par, head):
        for k in range(BLK // 16):
            sv = EB[par][pl.ds(k * 16, 16)]
            dv = EB[par][pl.ds(BLK + k * 16, 16)]
            sg = plsc.load_gather(apk_v, [sv])
            dg = plsc.load_gather(apk_v, [dv])
            asrc = plsc.bitcast(lax.shift_left(sg, 16), jnp.float32)
            adst = plsc.bitcast(jnp.bitwise_and(dg, himask), jnp.float32)
            e = asrc + adst
            e = jnp.where(e >= 0.0, e, 0.2 * e)
            ex = jnp.exp(e)
            WV[par][pl.ds(k * 16, 16)] = ex
            RIDX[par][pl.ds(k * 16, 16)] = sv * H + head
            DC[par][pl.ds(k * 16, 16)] = dv
            plsc.addupdate_scatter(denom_v, [dv], ex)

    def issue_gather(par):
        pltpu.async_copy(htable.at[RIDX[par]], ROWS[par], SEMG[par])

    def wait_gather(par):
        pltpu.make_async_copy(htable.at[RIDX[par]], ROWS[par],
                              SEMG[par]).wait()

    def scale(par):
        def g(k, _):
            w16 = WV[par][pl.ds(k * 16, 16)]
            for l in range(16):
                r = k * 16 + l
                wv = jnp.broadcast_to(w16[l], (16,))
                for j in range(C // 16):
                    ROWS[par][r, pl.ds(j * 16, 16)] = (
                        ROWS[par][r, pl.ds(j * 16, 16)] * wv)
            return 0
        lax.fori_loop(0, BLK // 16, g, 0)

    def issue_scatter(par):
        pltpu.async_copy(ROWS[par], acc.at[DC[par]], SEMS[par], add=True)

    def wait_scatter(par):
        pltpu.make_async_copy(ROWS[par], acc.at[DC[par]], SEMS[par]).wait()

    for p in range(2):
        head = c * 2 + p
        pltpu.sync_copy(
            alphasP.at[pl.ds(pl.multiple_of(head * N, 8), N)], apk_v)
        def _zb(i, _):
            for j in range(C // 16):
                rows0[i, pl.ds(j * 16, 16)] = zero16
            return 0
        lax.fori_loop(0, BLK, _zb, 0)
        for k in range(ROWS_PER_SUB // BLK):
            pltpu.sync_copy(rows0, acc.at[pl.ds(row0 + k * BLK, BLK)])
        zrem = ROWS_PER_SUB - (ROWS_PER_SUB // BLK) * BLK
        if zrem:
            pltpu.sync_copy(
                rows0.at[pl.ds(0, zrem)],
                acc.at[pl.ds(row0 + ROWS_PER_SUB - zrem, zrem)])
        def _zd(i, _):
            denom_v[pl.ds(i * 16, 16)] = zero16
            return 0
        lax.fori_loop(0, N // 16, _zd, 0)
        plsc.subcore_barrier()

        issue_idx(0, 0)
        wait_idx(0)
        weights(0, head)
        issue_gather(0)
        issue_idx(1, 1)

        def pair(i, _):
            b2 = 2 * i
            for par in (0, 1):
                b = b2 + par
                if par == 0:
                    wait_idx(1)

                    @pl.when(i > 0)
                    def _():
                        wait_scatter(1)
                    weights(1, head)
                    issue_gather(1)

                    @pl.when(i < NPAIR - 1)
                    def _():
                        issue_idx(b2 + 2, 0)
                    wait_gather(0)
                    scale(0)
                    issue_scatter(0)
                else:
                    @pl.when(i < NPAIR - 1)
                    def _():
                        wait_idx(0)
                    wait_scatter(0)

                    @pl.when(i < NPAIR - 1)
                    def _():
                        weights(0, head)
                        issue_gather(0)
                        issue_idx(b2 + 3, 1)
                    wait_gather(1)
                    scale(1)
                    issue_scatter(1)
            return 0
        lax.fori_loop(0, NPAIR, pair, 0)
        wait_scatter(1)

        toff = pl.multiple_of(2 * E_MAIN + s * (2 * TAIL), 8)
        pltpu.sync_copy(e2.at[pl.ds(toff, 2 * TAIL)], teb)

        for k in range(TAIL // 16):
            sv = teb[pl.ds(k * 16, 16)]
            dv = teb[pl.ds(TAIL + k * 16, 16)]
            sg = plsc.load_gather(apk_v, [sv])
            dg = plsc.load_gather(apk_v, [dv])
            asrc = plsc.bitcast(lax.shift_left(sg, 16), jnp.float32)
            adst = plsc.bitcast(jnp.bitwise_and(dg, himask), jnp.float32)
            e = asrc + adst
            e = jnp.where(e >= 0.0, e, 0.2 * e)
            ex = jnp.exp(e)
            tw[pl.ds(k * 16, 16)] = ex
            tridx[pl.ds(k * 16, 16)] = sv * H + head
            tdc[pl.ds(k * 16, 16)] = dv
            plsc.addupdate_scatter(denom_v, [dv], ex)
        pltpu.sync_copy(htable.at[tridx], rows0.at[pl.ds(0, TAIL)])

        def _tscale(k, _):
            w16 = tw[pl.ds(k * 16, 16)]
            for l in range(16):
                r = k * 16 + l
                wv = jnp.broadcast_to(w16[l], (16,))
                for j in range(C // 16):
                    rows0[r, pl.ds(j * 16, 16)] = (
                        rows0[r, pl.ds(j * 16, 16)] * wv)
            return 0
        lax.fori_loop(0, TAIL // 16, _tscale, 0)
        pltpu.sync_copy(rows0.at[pl.ds(0, TAIL)], acc.at[tdc], add=True)

        plsc.subcore_barrier()

        pltpu.sync_copy(acc.at[pl.ds(row0, ROWS_PER_SUB)],
                        msg_out.at[head, pl.ds(row0, ROWS_PER_SUB)])
        dbase = pl.multiple_of((head * NSUB + s) * N, 8)
        pltpu.sync_copy(denom_v, denomp_out.at[pl.ds(dbase, N)])


def _stage2(htable, e2, alphasP):
    mesh = plsc.VectorSubcoreMesh(core_axis_name="c", subcore_axis_name="s")
    kern = functools.partial(
        pl.kernel,
        out_type=[
            jax.ShapeDtypeStruct((H, NPAD, C), jnp.float32),
            jax.ShapeDtypeStruct((H * NSUB * N,), jnp.float32),
        ],
        mesh=mesh,
        scratch_types=(
            [pltpu.VMEM_SHARED((NPAD, C), jnp.float32)]
            + [pltpu.VMEM((N,), jnp.int32)]
            + [pltpu.VMEM((N,), jnp.float32)]
            + [pltpu.VMEM((2 * BLK,), jnp.int32)] * 2
            + [pltpu.VMEM((BLK,), jnp.int32)] * 2
            + [pltpu.VMEM((BLK,), jnp.float32)] * 2
            + [pltpu.VMEM((BLK,), jnp.int32)] * 2
            + [pltpu.VMEM((BLK, C), jnp.float32)] * 2
            + [pltpu.VMEM((2 * TAIL,), jnp.int32)]
            + [pltpu.VMEM((TAIL,), jnp.int32)]
            + [pltpu.VMEM((TAIL,), jnp.float32)]
            + [pltpu.VMEM((TAIL,), jnp.int32)]
            + [pltpu.SemaphoreType.DMA] * 6
        ),
        compiler_params=pltpu.CompilerParams(needs_layout_passes=False),
    )(_sc_body)
    return kern(htable, e2, alphasP)


def _stage3_body(msg_ref, denomp_ref, w1_ref, b1_ref, w2_ref, b2_ref,
                 w3_ref, b3_ref, enc_ref, t1_ref, t2_ref, t3_ref):
    denom = jnp.sum(denomp_ref[...], axis=1) + 1e-16
    msg = msg_ref[...]
    cols = []
    for h in range(H):
        mh = msg[h] / denom[h][:, None]
        eh = jnp.where(mh > 0.0, mh, jnp.exp(mh) - 1.0)
        cols.append(eh)
        enc_ref[:, h * C:(h + 1) * C] = eh
    enc = jnp.concatenate(cols, axis=1)
    t1_ref[...] = jnp.dot(enc, w1_ref[...],
                          preferred_element_type=jnp.float32) + b1_ref[...]
    t2_ref[...] = jnp.dot(enc, w2_ref[...],
                          preferred_element_type=jnp.float32) + b2_ref[...]
    t3_ref[...] = jnp.dot(enc, w3_ref[...],
                          preferred_element_type=jnp.float32) + b3_ref[...]


def _stage3(msg, denomp, W1, b1, W2, b2, W3, b3):
    blk = 1024
    grid = pl.cdiv(N, blk)
    d1, d2, d3 = W1.shape[1], W2.shape[1], W3.shape[1]
    return pl.pallas_call(
        _stage3_body,
        grid=(grid,),
        in_specs=[
            pl.BlockSpec((H, blk, C), lambda i: (0, i, 0)),
            pl.BlockSpec((H, NSUB, blk), lambda i: (0, 0, i)),
            pl.BlockSpec((DE, d1), lambda i: (0, 0)),
            pl.BlockSpec((1, d1), lambda i: (0, 0)),
            pl.BlockSpec((DE, d2), lambda i: (0, 0)),
            pl.BlockSpec((1, d2), lambda i: (0, 0)),
            pl.BlockSpec((DE, d3), lambda i: (0, 0)),
            pl.BlockSpec((1, d3), lambda i: (0, 0)),
        ],
        out_specs=[
            pl.BlockSpec((blk, DE), lambda i: (i, 0)),
            pl.BlockSpec((blk, d1), lambda i: (i, 0)),
            pl.BlockSpec((blk, d2), lambda i: (i, 0)),
            pl.BlockSpec((blk, d3), lambda i: (i, 0)),
        ],
        out_shape=[
            jax.ShapeDtypeStruct((N, DE), jnp.float32),
            jax.ShapeDtypeStruct((N, d1), jnp.float32),
            jax.ShapeDtypeStruct((N, d2), jnp.float32),
            jax.ShapeDtypeStruct((N, d3), jnp.float32),
        ],
    )(msg, denomp, W1, b1, W2, b2, W3, b3)


@jax.jit
def kernel(x, edge_index, W, a_src, a_dst, W1, b1, W2, b2, W3, b3):
    eye = jnp.eye(H, dtype=jnp.float32)
    a_s = (a_src[:, :, None] * eye[:, None, :]).reshape(DE, H)
    a_d = (a_dst[:, :, None] * eye[:, None, :]).reshape(DE, H)
    A = jnp.concatenate([a_s, a_d], axis=1)

    h, alphas = _stage1(x, W, A)
    asrc_u = lax.bitcast_convert_type(
        alphas[:, :H].astype(jnp.bfloat16), jnp.uint16).astype(jnp.uint32)
    adst_u = lax.bitcast_convert_type(
        alphas[:, H:].astype(jnp.bfloat16), jnp.uint16).astype(jnp.uint32)
    alphasP = lax.bitcast_convert_type(
        ((adst_u << 16) | asrc_u).T.reshape(-1), jnp.int32)
    htable = h.reshape(N * H, C)
    srcs = edge_index[0]
    dsts = edge_index[1]
    e2_main = jnp.stack(
        [srcs[:E_MAIN].reshape(-1, BLK), dsts[:E_MAIN].reshape(-1, BLK)],
        axis=1).reshape(-1)
    e2_tail = jnp.stack(
        [srcs[E_MAIN:].reshape(-1, TAIL), dsts[E_MAIN:].reshape(-1, TAIL)],
        axis=1).reshape(-1)
    e2 = jnp.concatenate([e2_main, e2_tail])
    msg, denomp = _stage2(htable, e2, alphasP)
    denomp = denomp.reshape(H, NSUB, N)
    enc, t1, t2, t3 = _stage3(msg, denomp, W1, b1.reshape(1, -1),
                              W2, b2.reshape(1, -1), W3, b3.reshape(1, -1))
    return (enc, t1, t2, t3)
